# baseline (speedup 1.0000x reference)
import functools

import jax
import jax.numpy as jnp
from jax import lax
from jax.experimental import pallas as pl
from jax.experimental.pallas import tpu as pltpu
from jax.experimental.pallas import tpu_sc as plsc


def _make_sc_kernel(N, D, n_workers, chunk):
    per_w = N // n_workers
    n_chunks = per_w // chunk
    mesh = plsc.VectorSubcoreMesh(core_axis_name="c", subcore_axis_name="s")

    @functools.partial(
        pl.kernel,
        mesh=mesh,
        compiler_params=pltpu.CompilerParams(use_tc_tiling_on_sc=False),
        out_type=jax.ShapeDtypeStruct((N, D), jnp.float32),
        scratch_types=[
            pltpu.VMEM((chunk,), jnp.int32),
            pltpu.VMEM((chunk,), jnp.int32),
            pltpu.VMEM((chunk, D), jnp.float32),
            pltpu.SemaphoreType.DMA,
            pltpu.SemaphoreType.DMA,
        ],
    )
    def k(words_hbm, w2w_hbm, emb_hbm, out_hbm, idx_v, remap_v, rows_v,
          sem1, sem2):
        wid = lax.axis_index("s") * 2 + lax.axis_index("c")
        base = wid * per_w

        def body(g, carry):
            off = base + g * chunk
            pltpu.sync_copy(words_hbm.at[pl.ds(off, chunk)], idx_v)
            pltpu.async_copy(w2w_hbm.at[idx_v], remap_v, sem1).wait()
            pltpu.async_copy(emb_hbm.at[remap_v], rows_v, sem2).wait()
            pltpu.sync_copy(rows_v, out_hbm.at[pl.ds(off, chunk)])
            return carry

        lax.fori_loop(0, n_chunks, body, 0)

    return k


def kernel(words, words_to_words, embedding_weight):
    B, L = words.shape
    D = embedding_weight.shape[1]
    N = B * L
    words_flat = words.reshape(N).astype(jnp.int32)
    k = _make_sc_kernel(N, D, n_workers=32, chunk=1600)
    out = k(words_flat, words_to_words, embedding_weight)
    return out.reshape(B, L, D)


# speedup vs baseline: 1.0254x; 1.0254x over previous
import functools

import jax
import jax.numpy as jnp
from jax import lax
from jax.experimental import pallas as pl
from jax.experimental.pallas import tpu as pltpu
from jax.experimental.pallas import tpu_sc as plsc


def _make_sc_kernel(N, D, n_workers, chunk):
    per_w = N // n_workers
    n_chunks = per_w // chunk
    assert n_chunks % 2 == 0 and n_chunks >= 4
    mesh = plsc.VectorSubcoreMesh(core_axis_name="c", subcore_axis_name="s")

    @functools.partial(
        pl.kernel,
        mesh=mesh,
        compiler_params=pltpu.CompilerParams(use_tc_tiling_on_sc=False),
        out_type=jax.ShapeDtypeStruct((N, D), jnp.float32),
        scratch_types=[
            pltpu.VMEM((2, chunk), jnp.int32),
            pltpu.VMEM((2, chunk), jnp.int32),
            pltpu.VMEM((2, chunk, D), jnp.float32),
            pltpu.SemaphoreType.DMA,
            pltpu.SemaphoreType.DMA,
            pltpu.SemaphoreType.DMA,
            pltpu.SemaphoreType.DMA,
            pltpu.SemaphoreType.DMA,
            pltpu.SemaphoreType.DMA,
        ],
    )
    def k(words_hbm, w2w_hbm, emb_hbm, out_hbm, idx_v, remap_v, rows_v,
          rs0, rs1, gs0, gs1, os0, os1):
        wid = lax.axis_index("s") * 2 + lax.axis_index("c")
        base = wid * per_w
        rsem = (rs0, rs1)
        gsem = (gs0, gs1)
        osem = (os0, os1)

        for p in (0, 1):
            pltpu.sync_copy(words_hbm.at[pl.ds(base + p * chunk, chunk)],
                            idx_v.at[p])
            pltpu.async_copy(w2w_hbm.at[idx_v.at[p]], remap_v.at[p], rsem[p])

        def pair(i, carry):
            for p in (0, 1):
                g = i * 2 + p
                off = base + g * chunk

                @pl.when(g >= 2)
                def _drain_old_write():
                    pltpu.make_async_copy(
                        rows_v.at[p],
                        out_hbm.at[pl.ds(off - 2 * chunk, chunk)],
                        osem[p],
                    ).wait()

                pltpu.make_async_copy(
                    w2w_hbm.at[idx_v.at[p]], remap_v.at[p], rsem[p]
                ).wait()
                pltpu.async_copy(
                    emb_hbm.at[remap_v.at[p]], rows_v.at[p], gsem[p]
                ).wait()
                pltpu.async_copy(
                    rows_v.at[p], out_hbm.at[pl.ds(off, chunk)], osem[p]
                )

                @pl.when(g + 2 < n_chunks)
                def _prefetch_remap():
                    pltpu.sync_copy(
                        words_hbm.at[pl.ds(off + 2 * chunk, chunk)],
                        idx_v.at[p],
                    )
                    pltpu.async_copy(
                        w2w_hbm.at[idx_v.at[p]], remap_v.at[p], rsem[p]
                    )

            return carry

        lax.fori_loop(0, n_chunks // 2, pair, 0)

        for p in (0, 1):
            g = n_chunks - 2 + p
            pltpu.make_async_copy(
                rows_v.at[p],
                out_hbm.at[pl.ds(base + g * chunk, chunk)],
                osem[p],
            ).wait()

    return k


def kernel(words, words_to_words, embedding_weight):
    B, L = words.shape
    D = embedding_weight.shape[1]
    N = B * L
    words_flat = words.reshape(N).astype(jnp.int32)
    k = _make_sc_kernel(N, D, n_workers=32, chunk=800)
    out = k(words_flat, words_to_words, embedding_weight)
    return out.reshape(B, L, D)


# speedup vs baseline: 1.1111x; 1.0836x over previous
import functools

import jax
import jax.numpy as jnp
from jax import lax
from jax.experimental import pallas as pl
from jax.experimental.pallas import tpu as pltpu
from jax.experimental.pallas import tpu_sc as plsc

_DP = 128


def _make_sc_kernel(N, D, n_workers, chunk):
    per_w = N // n_workers
    n_chunks = per_w // chunk
    assert n_chunks % 2 == 0 and n_chunks >= 4
    mesh = plsc.VectorSubcoreMesh(core_axis_name="c", subcore_axis_name="s")

    @functools.partial(
        pl.kernel,
        mesh=mesh,
        out_type=jax.ShapeDtypeStruct((N, _DP), jnp.float32),
        scratch_types=[
            pltpu.VMEM((2 * chunk,), jnp.int32),
            pltpu.VMEM((2 * chunk,), jnp.int32),
            pltpu.VMEM((2, chunk, _DP), jnp.float32),
            pltpu.SemaphoreType.DMA,
            pltpu.SemaphoreType.DMA,
            pltpu.SemaphoreType.DMA,
            pltpu.SemaphoreType.DMA,
            pltpu.SemaphoreType.DMA,
            pltpu.SemaphoreType.DMA,
        ],
    )
    def k(words_hbm, w2w_hbm, emb_hbm, out_hbm, idx_v, remap_v, rows_v,
          rs0, rs1, gs0, gs1, os0, os1):
        wid = lax.axis_index("s") * 2 + lax.axis_index("c")
        base = wid * per_w
        rsem = (rs0, rs1)
        gsem = (gs0, gs1)
        osem = (os0, os1)

        for p in (0, 1):
            pltpu.sync_copy(words_hbm.at[pl.ds(base + p * chunk, chunk)],
                            idx_v.at[pl.ds(p * chunk, chunk)])
            pltpu.async_copy(w2w_hbm.at[idx_v.at[pl.ds(p * chunk, chunk)]], remap_v.at[pl.ds(p * chunk, chunk)], rsem[p])

        def pair(i, carry):
            for p in (0, 1):
                g = i * 2 + p
                off = base + g * chunk

                @pl.when(g >= 2)
                def _drain_old_write():
                    pltpu.make_async_copy(
                        rows_v.at[p],
                        out_hbm.at[pl.ds(off - 2 * chunk, chunk)],
                        osem[p],
                    ).wait()

                pltpu.make_async_copy(
                    w2w_hbm.at[idx_v.at[pl.ds(p * chunk, chunk)]], remap_v.at[pl.ds(p * chunk, chunk)], rsem[p]
                ).wait()
                pltpu.async_copy(
                    emb_hbm.at[remap_v.at[pl.ds(p * chunk, chunk)]], rows_v.at[p], gsem[p]
                ).wait()
                pltpu.async_copy(
                    rows_v.at[p],
                    out_hbm.at[pl.ds(off, chunk)],
                    osem[p],
                )

                @pl.when(g + 2 < n_chunks)
                def _prefetch_remap():
                    pltpu.sync_copy(
                        words_hbm.at[pl.ds(off + 2 * chunk, chunk)],
                        idx_v.at[pl.ds(p * chunk, chunk)],
                    )
                    pltpu.async_copy(
                        w2w_hbm.at[idx_v.at[pl.ds(p * chunk, chunk)]], remap_v.at[pl.ds(p * chunk, chunk)], rsem[p]
                    )

            return carry

        lax.fori_loop(0, n_chunks // 2, pair, 0)

        for p in (0, 1):
            g = n_chunks - 2 + p
            pltpu.make_async_copy(
                rows_v.at[p],
                out_hbm.at[pl.ds(base + g * chunk, chunk)],
                osem[p],
            ).wait()

    return k


def kernel(words, words_to_words, embedding_weight):
    B, L = words.shape
    D = embedding_weight.shape[1]
    N = B * L
    words_flat = words.reshape(N).astype(jnp.int32)
    emb_padded = jnp.pad(embedding_weight, ((0, 0), (0, _DP - D)))
    k = _make_sc_kernel(N, D, n_workers=32, chunk=128)
    out = k(words_flat, words_to_words, emb_padded)
    return out[:, :D].reshape(B, L, D)


# speedup vs baseline: 1.4541x; 1.3086x over previous
import functools

import jax
import jax.numpy as jnp
from jax import lax
from jax.experimental import pallas as pl
from jax.experimental.pallas import tpu as pltpu
from jax.experimental.pallas import tpu_sc as plsc

_DP = 128


def _make_sc_kernel(N, D, n_workers, chunk):
    per_w = N // n_workers
    n_chunks = per_w // chunk
    assert n_chunks % 2 == 0 and n_chunks >= 4
    mesh = plsc.VectorSubcoreMesh(core_axis_name="c", subcore_axis_name="s")

    @functools.partial(
        pl.kernel,
        mesh=mesh,
        compiler_params=pltpu.CompilerParams(use_tc_tiling_on_sc=False),
        out_type=jax.ShapeDtypeStruct((N, _DP), jnp.float32),
        scratch_types=[
            pltpu.VMEM((2 * chunk,), jnp.int32),
            pltpu.VMEM((2 * chunk,), jnp.int32),
            pltpu.VMEM((2, chunk, D), jnp.float32),
            pltpu.SemaphoreType.DMA,
            pltpu.SemaphoreType.DMA,
            pltpu.SemaphoreType.DMA,
            pltpu.SemaphoreType.DMA,
            pltpu.SemaphoreType.DMA,
            pltpu.SemaphoreType.DMA,
        ],
    )
    def k(words_hbm, w2w_hbm, emb_hbm, out_hbm, idx_v, remap_v, rows_v,
          rs0, rs1, gs0, gs1, os0, os1):
        wid = lax.axis_index("s") * 2 + lax.axis_index("c")
        base = wid * per_w
        rsem = (rs0, rs1)
        gsem = (gs0, gs1)
        osem = (os0, os1)

        for p in (0, 1):
            pltpu.sync_copy(words_hbm.at[pl.ds(base + p * chunk, chunk)],
                            idx_v.at[pl.ds(p * chunk, chunk)])
            pltpu.async_copy(w2w_hbm.at[idx_v.at[pl.ds(p * chunk, chunk)]], remap_v.at[pl.ds(p * chunk, chunk)], rsem[p])

        def pair(i, carry):
            for p in (0, 1):
                g = i * 2 + p
                off = base + g * chunk

                @pl.when(g >= 2)
                def _drain_old_write():
                    pltpu.make_async_copy(
                        rows_v.at[p],
                        out_hbm.at[pl.ds(off - 2 * chunk, chunk), pl.ds(0, D)],
                        osem[p],
                    ).wait()

                pltpu.make_async_copy(
                    w2w_hbm.at[idx_v.at[pl.ds(p * chunk, chunk)]], remap_v.at[pl.ds(p * chunk, chunk)], rsem[p]
                ).wait()
                pltpu.async_copy(
                    emb_hbm.at[remap_v.at[pl.ds(p * chunk, chunk)]], rows_v.at[p], gsem[p]
                ).wait()
                pltpu.async_copy(
                    rows_v.at[p],
                    out_hbm.at[pl.ds(off, chunk), pl.ds(0, D)],
                    osem[p],
                )

                @pl.when(g + 2 < n_chunks)
                def _prefetch_remap():
                    pltpu.sync_copy(
                        words_hbm.at[pl.ds(off + 2 * chunk, chunk)],
                        idx_v.at[pl.ds(p * chunk, chunk)],
                    )
                    pltpu.async_copy(
                        w2w_hbm.at[idx_v.at[pl.ds(p * chunk, chunk)]], remap_v.at[pl.ds(p * chunk, chunk)], rsem[p]
                    )

            return carry

        lax.fori_loop(0, n_chunks // 2, pair, 0)

        for p in (0, 1):
            g = n_chunks - 2 + p
            pltpu.make_async_copy(
                rows_v.at[p],
                out_hbm.at[pl.ds(base + g * chunk, chunk), pl.ds(0, D)],
                osem[p],
            ).wait()

    return k


def kernel(words, words_to_words, embedding_weight):
    B, L = words.shape
    V, D = embedding_weight.shape
    N = B * L
    words_flat = words.reshape(N).astype(jnp.int32)
    emb_padded = jnp.pad(embedding_weight, ((0, 0), (0, _DP - D)))
    emb_rows = emb_padded.reshape(V * _DP // D, D)
    w2w2 = words_to_words * 2
    k = _make_sc_kernel(N, D, n_workers=32, chunk=800)
    out = k(words_flat, w2w2, emb_rows)
    return out[:, :D].reshape(B, L, D)
